# Initial kernel scaffold; baseline (speedup 1.0000x reference)
import jax
import jax.numpy as jnp
from jax.experimental import pallas as pl


def kernel(pos, atom_coords, ee_node_emb, ee_edge_emb, ee_Wrbf, ee_Wself, ee_Wout, en_node_emb, en_edge_emb, en_Wrbf, en_Wself, en_Wout):
    raise NotImplementedError("write your pallas kernel here")


# speedup vs baseline: 44.9397x; 44.9397x over previous
import numpy as np
import jax
import jax.numpy as jnp
from jax import lax
from jax.experimental import pallas as pl
from jax.experimental.pallas import tpu as pltpu

NELEC = 64
NUP = 32
NATOMS = 16
FEAT = 32
NRBF = 32
NLAYERS = 2
NBATCH = 256
GAMMA = 10.0
BB = 8

_CENTERS = np.linspace(0.0, 10.0, NRBF).astype(np.float32).reshape(1, NRBF)


def _body(x_ref, y_ref, z_ref, ax_ref, ay_ref, az_ref, cen_ref,
          een_ref, eeg_ref, eewr_ref, eews_ref, eewo_ref,
          enn_ref, enga_ref, enge_ref, enwr_ref, enws_ref, enwo_ref,
          out_ref):
    x = x_ref[...]
    y = y_ref[...]
    z = z_ref[...]
    c4 = cen_ref[...].reshape(1, 1, 1, NRBF)

    jmask = lax.broadcasted_iota(jnp.int32, (NELEC, FEAT), 0) >= NUP

    dx = x[:, :, None] - x[:, None, :]
    dy = y[:, :, None] - y[:, None, :]
    dz = z[:, :, None] - z[:, None, :]
    dee = jnp.sqrt(dx * dx + dy * dy + dz * dz + 1e-12)
    ii = lax.broadcasted_iota(jnp.int32, (BB, NELEC, NELEC), 1)
    jj = lax.broadcasted_iota(jnp.int32, (BB, NELEC, NELEC), 2)
    dee = jnp.where(ii == jj, 1.0e4, dee)
    rbf = jnp.exp(-GAMMA * (dee[..., None] - c4) ** 2)

    eeg = eeg_ref[...]
    g_up = jnp.where(jmask, eeg[1:2, :], eeg[0:1, :])
    g_dn = jnp.where(jmask, eeg[2:3, :], eeg[1:2, :])

    een = een_ref[...]
    h = jnp.broadcast_to(jnp.where(jmask, een[1:2, :], een[0:1, :])[None],
                         (BB, NELEC, FEAT))

    for l in range(NLAYERS):
        E = jnp.dot(rbf.reshape(BB * NELEC * NELEC, NRBF), eewr_ref[l],
                    preferred_element_type=jnp.float32)
        E = E.reshape(BB, NELEC, NELEC, FEAT)
        hg_up = g_up[None] * h
        hg_dn = g_dn[None] * h
        agg_top = jnp.sum(E[:, :NUP] * hg_up[:, None], axis=2)
        agg_bot = jnp.sum(E[:, NUP:] * hg_dn[:, None], axis=2)
        agg = jnp.concatenate([agg_top, agg_bot], axis=1)
        upd = jnp.dot(agg.reshape(BB * NELEC, FEAT), eews_ref[l],
                      preferred_element_type=jnp.float32)
        h = h + jnp.tanh(upd).reshape(BB, NELEC, FEAT)

    ee_k = jnp.sum(jnp.sum(h, axis=1) * eewo_ref[...], axis=1)

    ax = ax_ref[...].reshape(1, NATOMS, 1)
    ay = ay_ref[...].reshape(1, NATOMS, 1)
    az = az_ref[...].reshape(1, NATOMS, 1)
    dxa = ax - x[:, None, :]
    dya = ay - y[:, None, :]
    dza = az - z[:, None, :]
    den = jnp.sqrt(dxa * dxa + dya * dya + dza * dza + 1e-12)
    rbf_en = jnp.exp(-GAMMA * (den[..., None] - c4) ** 2)

    enn = enn_ref[...]
    he = jnp.broadcast_to(jnp.where(jmask, enn[1:2, :], enn[0:1, :])[None],
                          (BB, NELEC, FEAT))
    ha = jnp.broadcast_to(enn[2:2 + NATOMS][None], (BB, NATOMS, FEAT))
    ga = enga_ref[...]
    ge = enge_ref[...]

    for l in range(NLAYERS):
        Een = jnp.dot(rbf_en.reshape(BB * NATOMS * NELEC, NRBF), enwr_ref[l],
                      preferred_element_type=jnp.float32)
        Een = Een.reshape(BB, NATOMS, NELEC, FEAT)
        hag = (ge[None] * ha)[:, :, None, :]
        agg_e = jnp.sum(Een * hag, axis=1)
        t = jnp.sum(Een * he[:, None, :, :], axis=2)
        agg_a = ga[None] * t
        upd_e = jnp.dot(agg_e.reshape(BB * NELEC, FEAT), enws_ref[l],
                        preferred_element_type=jnp.float32)
        upd_a = jnp.dot(agg_a.reshape(BB * NATOMS, FEAT), enws_ref[l],
                        preferred_element_type=jnp.float32)
        he = he + jnp.tanh(upd_e).reshape(BB, NELEC, FEAT)
        ha = ha + jnp.tanh(upd_a).reshape(BB, NATOMS, FEAT)

    nsum = jnp.sum(he, axis=1) + jnp.sum(ha, axis=1)
    en_k = jnp.sum(nsum * enwo_ref[...], axis=1)

    out_ref[...] = jnp.exp(ee_k + en_k).reshape(1, 1, BB)


def kernel(pos, atom_coords, ee_node_emb, ee_edge_emb, ee_Wrbf, ee_Wself,
           ee_Wout, en_node_emb, en_edge_emb, en_Wrbf, en_Wself, en_Wout):
    nb = pos.shape[0]
    xyz = pos.reshape(nb, NELEC, 3)
    X = xyz[:, :, 0]
    Y = xyz[:, :, 1]
    Z = xyz[:, :, 2]
    AX = atom_coords[:, 0].reshape(1, NATOMS)
    AY = atom_coords[:, 1].reshape(1, NATOMS)
    AZ = atom_coords[:, 2].reshape(1, NATOMS)
    cen = jnp.asarray(_CENTERS)
    eeWoT = ee_Wout.reshape(1, FEAT)
    enWoT = en_Wout.reshape(1, FEAT)
    enGA = en_edge_emb[0::2]
    enGE = en_edge_emb[1::2]

    grid = nb // BB
    full = lambda shape: pl.BlockSpec(shape, lambda i, s=len(shape): (0,) * s)
    out = pl.pallas_call(
        _body,
        grid=(grid,),
        in_specs=[
            pl.BlockSpec((BB, NELEC), lambda i: (i, 0)),
            pl.BlockSpec((BB, NELEC), lambda i: (i, 0)),
            pl.BlockSpec((BB, NELEC), lambda i: (i, 0)),
            full((1, NATOMS)), full((1, NATOMS)), full((1, NATOMS)),
            full((1, NRBF)),
            full((2, FEAT)), full((3, FEAT)),
            full((NLAYERS, NRBF, FEAT)), full((NLAYERS, FEAT, FEAT)),
            full((1, FEAT)),
            full((2 + NATOMS, FEAT)),
            full((NATOMS, FEAT)), full((NATOMS, FEAT)),
            full((NLAYERS, NRBF, FEAT)), full((NLAYERS, FEAT, FEAT)),
            full((1, FEAT)),
        ],
        out_specs=pl.BlockSpec((1, 1, BB), lambda i: (i, 0, 0)),
        out_shape=jax.ShapeDtypeStruct((grid, 1, BB), jnp.float32),
        compiler_params=pltpu.CompilerParams(
            dimension_semantics=("parallel",)),
    )(X, Y, Z, AX, AY, AZ, cen,
      ee_node_emb, ee_edge_emb, ee_Wrbf, ee_Wself, eeWoT,
      en_node_emb, enGA, enGE, en_Wrbf, en_Wself, enWoT)
    return out.reshape(nb, 1)


# speedup vs baseline: 51.2042x; 1.1394x over previous
import numpy as np
import jax
import jax.numpy as jnp
from jax import lax
from jax.experimental import pallas as pl
from jax.experimental.pallas import tpu as pltpu

NELEC = 64
NUP = 32
NATOMS = 16
FEAT = 32
NRBF = 32
NLAYERS = 2
NBATCH = 256
GAMMA = 10.0
BB = 8

_CENTERS = np.linspace(0.0, 10.0, NRBF).astype(np.float32).reshape(1, NRBF)


def _body(x_ref, y_ref, z_ref, ax_ref, ay_ref, az_ref, cen_ref,
          een_ref, eeg_ref, eewr_ref, eews_ref, eewo_ref,
          enn_ref, enga_ref, enge_ref, enwr_ref, enws_ref, enwo_ref,
          out_ref):
    x = x_ref[...]
    y = y_ref[...]
    z = z_ref[...]
    c4 = cen_ref[...].reshape(1, 1, 1, NRBF)

    jmask = lax.broadcasted_iota(jnp.int32, (NELEC, FEAT), 0) >= NUP

    dx = x[:, :, None] - x[:, None, :]
    dy = y[:, :, None] - y[:, None, :]
    dz = z[:, :, None] - z[:, None, :]
    dee = jnp.sqrt(dx * dx + dy * dy + dz * dz + 1e-12)
    ii = lax.broadcasted_iota(jnp.int32, (BB, NELEC, NELEC), 1)
    jj = lax.broadcasted_iota(jnp.int32, (BB, NELEC, NELEC), 2)
    dee = jnp.where(ii == jj, 1.0e4, dee)
    rbf = jnp.exp(-GAMMA * (dee[..., None] - c4) ** 2)

    eeg = eeg_ref[...]
    g_up = jnp.where(jmask, eeg[1:2, :], eeg[0:1, :])
    g_dn = jnp.where(jmask, eeg[2:3, :], eeg[1:2, :])

    een = een_ref[...]
    h = jnp.broadcast_to(jnp.where(jmask, een[1:2, :], een[0:1, :])[None],
                         (BB, NELEC, FEAT))

    R2 = rbf.reshape(BB, NELEC, NELEC * NRBF)
    for l in range(NLAYERS):
        Wcat = jnp.concatenate([eewr_ref[l], eewr_ref[l]], axis=1)
        HG = jnp.concatenate([g_up[None] * h, g_dn[None] * h], axis=2)
        B2 = (HG[:, :, None, :] * Wcat[None, None, :, :]
              ).reshape(BB, NELEC * NRBF, 2 * FEAT)
        AG = lax.dot_general(R2, B2, (((2,), (1,)), ((0,), (0,))),
                             preferred_element_type=jnp.float32)
        agg = jnp.concatenate([AG[:, :NUP, :FEAT], AG[:, NUP:, FEAT:]],
                              axis=1)
        upd = jnp.dot(agg.reshape(BB * NELEC, FEAT), eews_ref[l],
                      preferred_element_type=jnp.float32)
        h = h + jnp.tanh(upd).reshape(BB, NELEC, FEAT)

    ee_k = jnp.sum(jnp.sum(h, axis=1) * eewo_ref[...], axis=1)

    ax = ax_ref[...].reshape(1, NATOMS, 1)
    ay = ay_ref[...].reshape(1, NATOMS, 1)
    az = az_ref[...].reshape(1, NATOMS, 1)
    dxa = ax - x[:, None, :]
    dya = ay - y[:, None, :]
    dza = az - z[:, None, :]
    den = jnp.sqrt(dxa * dxa + dya * dya + dza * dza + 1e-12)
    rbf_en = jnp.exp(-GAMMA * (den[..., None] - c4) ** 2)

    enn = enn_ref[...]
    he = jnp.broadcast_to(jnp.where(jmask, enn[1:2, :], enn[0:1, :])[None],
                          (BB, NELEC, FEAT))
    ha = jnp.broadcast_to(enn[2:2 + NATOMS][None], (BB, NATOMS, FEAT))
    ga = enga_ref[...]
    ge = enge_ref[...]

    for l in range(NLAYERS):
        Een = jnp.dot(rbf_en.reshape(BB * NATOMS * NELEC, NRBF), enwr_ref[l],
                      preferred_element_type=jnp.float32)
        Een = Een.reshape(BB, NATOMS, NELEC, FEAT)
        hag = (ge[None] * ha)[:, :, None, :]
        agg_e = jnp.sum(Een * hag, axis=1)
        t = jnp.sum(Een * he[:, None, :, :], axis=2)
        agg_a = ga[None] * t
        upd_e = jnp.dot(agg_e.reshape(BB * NELEC, FEAT), enws_ref[l],
                        preferred_element_type=jnp.float32)
        upd_a = jnp.dot(agg_a.reshape(BB * NATOMS, FEAT), enws_ref[l],
                        preferred_element_type=jnp.float32)
        he = he + jnp.tanh(upd_e).reshape(BB, NELEC, FEAT)
        ha = ha + jnp.tanh(upd_a).reshape(BB, NATOMS, FEAT)

    nsum = jnp.sum(he, axis=1) + jnp.sum(ha, axis=1)
    en_k = jnp.sum(nsum * enwo_ref[...], axis=1)

    out_ref[...] = jnp.exp(ee_k + en_k).reshape(1, 1, BB)


def kernel(pos, atom_coords, ee_node_emb, ee_edge_emb, ee_Wrbf, ee_Wself,
           ee_Wout, en_node_emb, en_edge_emb, en_Wrbf, en_Wself, en_Wout):
    nb = pos.shape[0]
    xyz = pos.reshape(nb, NELEC, 3)
    X = xyz[:, :, 0]
    Y = xyz[:, :, 1]
    Z = xyz[:, :, 2]
    AX = atom_coords[:, 0].reshape(1, NATOMS)
    AY = atom_coords[:, 1].reshape(1, NATOMS)
    AZ = atom_coords[:, 2].reshape(1, NATOMS)
    cen = jnp.asarray(_CENTERS)
    eeWoT = ee_Wout.reshape(1, FEAT)
    enWoT = en_Wout.reshape(1, FEAT)
    enGA = en_edge_emb[0::2]
    enGE = en_edge_emb[1::2]

    grid = nb // BB
    full = lambda shape: pl.BlockSpec(shape, lambda i, s=len(shape): (0,) * s)
    out = pl.pallas_call(
        _body,
        grid=(grid,),
        in_specs=[
            pl.BlockSpec((BB, NELEC), lambda i: (i, 0)),
            pl.BlockSpec((BB, NELEC), lambda i: (i, 0)),
            pl.BlockSpec((BB, NELEC), lambda i: (i, 0)),
            full((1, NATOMS)), full((1, NATOMS)), full((1, NATOMS)),
            full((1, NRBF)),
            full((2, FEAT)), full((3, FEAT)),
            full((NLAYERS, NRBF, FEAT)), full((NLAYERS, FEAT, FEAT)),
            full((1, FEAT)),
            full((2 + NATOMS, FEAT)),
            full((NATOMS, FEAT)), full((NATOMS, FEAT)),
            full((NLAYERS, NRBF, FEAT)), full((NLAYERS, FEAT, FEAT)),
            full((1, FEAT)),
        ],
        out_specs=pl.BlockSpec((1, 1, BB), lambda i: (i, 0, 0)),
        out_shape=jax.ShapeDtypeStruct((grid, 1, BB), jnp.float32),
        compiler_params=pltpu.CompilerParams(
            dimension_semantics=("parallel",)),
    )(X, Y, Z, AX, AY, AZ, cen,
      ee_node_emb, ee_edge_emb, ee_Wrbf, ee_Wself, eeWoT,
      en_node_emb, enGA, enGE, en_Wrbf, en_Wself, enWoT)
    return out.reshape(nb, 1)


# speedup vs baseline: 72.4306x; 1.4145x over previous
import numpy as np
import jax
import jax.numpy as jnp
from jax import lax
from jax.experimental import pallas as pl
from jax.experimental.pallas import tpu as pltpu

NELEC = 64
NUP = 32
NATOMS = 16
FEAT = 32
NRBF = 32
NLAYERS = 2
NBATCH = 256
GAMMA = 10.0
BB = 8

_CENTERS = np.linspace(0.0, 10.0, NRBF).astype(np.float32).reshape(1, NRBF)


def _body(x_ref, y_ref, z_ref, ax_ref, ay_ref, az_ref, cen_ref,
          een_ref, eeg_ref, eewr_ref, eews_ref, eewo_ref,
          enn_ref, enga_ref, enge_ref, enwr_ref, enws_ref, enwo_ref,
          out_ref):
    x = x_ref[...]
    y = y_ref[...]
    z = z_ref[...]
    cen = cen_ref[...]
    c4 = cen.reshape(1, 1, 1, NRBF)
    rbf0 = jnp.exp(-GAMMA * (1e-6 - cen) ** 2)

    jmask = lax.broadcasted_iota(jnp.int32, (NELEC, FEAT), 0) >= NUP

    dx = x[:, :, None] - x[:, None, :]
    dy = y[:, :, None] - y[:, None, :]
    dz = z[:, :, None] - z[:, None, :]
    dee = jnp.sqrt(dx * dx + dy * dy + dz * dz + 1e-12)
    ct = cen.reshape(1, 1, NRBF, 1)
    rbf = jnp.exp(-GAMMA * (dee[:, :, None, :] - ct) ** 2)
    R2 = rbf.reshape(BB, NELEC, NRBF * NELEC)

    eeg = eeg_ref[...]
    g_up = jnp.where(jmask, eeg[1:2, :], eeg[0:1, :])
    g_dn = jnp.where(jmask, eeg[2:3, :], eeg[1:2, :])

    een = een_ref[...]
    h0 = jnp.where(jmask, een[1:2, :], een[0:1, :])

    rsel = lax.broadcasted_iota(jnp.int32, (NRBF * NELEC, 2 * NRBF), 0)
    csel = lax.broadcasted_iota(jnp.int32, (NRBF * NELEC, 2 * NRBF), 1)
    Sel = jnp.where((rsel // NELEC == csel % NRBF)
                    & ((rsel % NELEC >= NUP) == (csel // NRBF == 1)),
                    1.0, 0.0)
    PQ = lax.dot_general(R2, Sel, (((2,), (0,)), ((), ())),
                         preferred_element_type=jnp.float32)

    a_lo = jnp.concatenate([eeg[0:1] * een[0:1], eeg[1:2] * een[0:1]], 1)
    a_hi = jnp.concatenate([eeg[1:2] * een[1:2], eeg[2:3] * een[1:2]], 1)
    imask2 = lax.broadcasted_iota(jnp.int32, (NELEC, 2 * FEAT), 0) >= NUP
    HG0 = jnp.where(imask2, a_hi, a_lo)

    h = jnp.broadcast_to(h0[None], (BB, NELEC, FEAT))
    delta = None
    for l in range(NLAYERS):
        Wcat = jnp.concatenate([eewr_ref[l], eewr_ref[l]], axis=1)
        rbf0W = jnp.sum(rbf0.reshape(NRBF, 1) * Wcat, axis=0,
                        keepdims=True)
        Wstk = jnp.concatenate([Wcat * a_lo, Wcat * a_hi], axis=0)
        AG = lax.dot_general(PQ, Wstk, (((2,), (0,)), ((), ())),
                             preferred_element_type=jnp.float32)
        AG = AG - (rbf0W * HG0)[None]
        if delta is not None:
            HGd = jnp.concatenate([g_up[None] * delta, g_dn[None] * delta],
                                  axis=2)
            B2 = (HGd[:, None, :, :] * Wcat.reshape(1, NRBF, 1, 2 * FEAT)
                  ).reshape(BB, NRBF * NELEC, 2 * FEAT)
            AG = AG + lax.dot_general(R2, B2, (((2,), (1,)), ((0,), (0,))),
                                      preferred_element_type=jnp.float32)
            AG = AG - rbf0W[None] * HGd
        agg = jnp.concatenate([AG[:, :NUP, :FEAT], AG[:, NUP:, FEAT:]],
                              axis=1)
        upd = jnp.dot(agg.reshape(BB * NELEC, FEAT), eews_ref[l],
                      preferred_element_type=jnp.float32)
        d_new = jnp.tanh(upd).reshape(BB, NELEC, FEAT)
        delta = d_new if delta is None else delta + d_new
        h = h + d_new

    ee_k = jnp.sum(jnp.sum(h, axis=1) * eewo_ref[...], axis=1)

    ax = ax_ref[...].reshape(1, NATOMS, 1)
    ay = ay_ref[...].reshape(1, NATOMS, 1)
    az = az_ref[...].reshape(1, NATOMS, 1)
    dxa = ax - x[:, None, :]
    dya = ay - y[:, None, :]
    dza = az - z[:, None, :]
    den = jnp.sqrt(dxa * dxa + dya * dya + dza * dza + 1e-12)
    rbf_en = jnp.exp(-GAMMA * (den[..., None] - c4) ** 2)

    enn = enn_ref[...]
    he = jnp.broadcast_to(jnp.where(jmask, enn[1:2, :], enn[0:1, :])[None],
                          (BB, NELEC, FEAT))
    ha = jnp.broadcast_to(enn[2:2 + NATOMS][None], (BB, NATOMS, FEAT))
    ga = enga_ref[...]
    ge = enge_ref[...]

    for l in range(NLAYERS):
        Een = jnp.dot(rbf_en.reshape(BB * NATOMS * NELEC, NRBF), enwr_ref[l],
                      preferred_element_type=jnp.float32)
        Een = Een.reshape(BB, NATOMS, NELEC, FEAT)
        hag = (ge[None] * ha)[:, :, None, :]
        agg_e = jnp.sum(Een * hag, axis=1)
        t = jnp.sum(Een * he[:, None, :, :], axis=2)
        agg_a = ga[None] * t
        upd_e = jnp.dot(agg_e.reshape(BB * NELEC, FEAT), enws_ref[l],
                        preferred_element_type=jnp.float32)
        upd_a = jnp.dot(agg_a.reshape(BB * NATOMS, FEAT), enws_ref[l],
                        preferred_element_type=jnp.float32)
        he = he + jnp.tanh(upd_e).reshape(BB, NELEC, FEAT)
        ha = ha + jnp.tanh(upd_a).reshape(BB, NATOMS, FEAT)

    nsum = jnp.sum(he, axis=1) + jnp.sum(ha, axis=1)
    en_k = jnp.sum(nsum * enwo_ref[...], axis=1)

    out_ref[...] = jnp.exp(ee_k + en_k).reshape(1, 1, BB)


def kernel(pos, atom_coords, ee_node_emb, ee_edge_emb, ee_Wrbf, ee_Wself,
           ee_Wout, en_node_emb, en_edge_emb, en_Wrbf, en_Wself, en_Wout):
    nb = pos.shape[0]
    xyz = pos.reshape(nb, NELEC, 3)
    X = xyz[:, :, 0]
    Y = xyz[:, :, 1]
    Z = xyz[:, :, 2]
    AX = atom_coords[:, 0].reshape(1, NATOMS)
    AY = atom_coords[:, 1].reshape(1, NATOMS)
    AZ = atom_coords[:, 2].reshape(1, NATOMS)
    cen = jnp.asarray(_CENTERS)
    eeWoT = ee_Wout.reshape(1, FEAT)
    enWoT = en_Wout.reshape(1, FEAT)
    enGA = en_edge_emb[0::2]
    enGE = en_edge_emb[1::2]

    grid = nb // BB
    full = lambda shape: pl.BlockSpec(shape, lambda i, s=len(shape): (0,) * s)
    out = pl.pallas_call(
        _body,
        grid=(grid,),
        in_specs=[
            pl.BlockSpec((BB, NELEC), lambda i: (i, 0)),
            pl.BlockSpec((BB, NELEC), lambda i: (i, 0)),
            pl.BlockSpec((BB, NELEC), lambda i: (i, 0)),
            full((1, NATOMS)), full((1, NATOMS)), full((1, NATOMS)),
            full((1, NRBF)),
            full((2, FEAT)), full((3, FEAT)),
            full((NLAYERS, NRBF, FEAT)), full((NLAYERS, FEAT, FEAT)),
            full((1, FEAT)),
            full((2 + NATOMS, FEAT)),
            full((NATOMS, FEAT)), full((NATOMS, FEAT)),
            full((NLAYERS, NRBF, FEAT)), full((NLAYERS, FEAT, FEAT)),
            full((1, FEAT)),
        ],
        out_specs=pl.BlockSpec((1, 1, BB), lambda i: (i, 0, 0)),
        out_shape=jax.ShapeDtypeStruct((grid, 1, BB), jnp.float32),
        compiler_params=pltpu.CompilerParams(
            dimension_semantics=("parallel",)),
    )(X, Y, Z, AX, AY, AZ, cen,
      ee_node_emb, ee_edge_emb, ee_Wrbf, ee_Wself, eeWoT,
      en_node_emb, enGA, enGE, en_Wrbf, en_Wself, enWoT)
    return out.reshape(nb, 1)


# speedup vs baseline: 127.1276x; 1.7552x over previous
import numpy as np
import jax
import jax.numpy as jnp
from jax import lax
from jax.experimental import pallas as pl
from jax.experimental.pallas import tpu as pltpu

NELEC = 64
NUP = 32
NATOMS = 16
FEAT = 32
NRBF = 32
NLAYERS = 2
NBATCH = 256
GAMMA = 10.0
BB = 8

_CENTERS = np.linspace(0.0, 10.0, NRBF).astype(np.float32).reshape(1, NRBF)
_CENTERS_EE = np.repeat(_CENTERS.ravel(), NELEC).reshape(1, NRBF * NELEC)
_CENTERS_EN = np.repeat(_CENTERS.ravel(), NATOMS).reshape(1, NRBF * NATOMS)


def _body(x_ref, y_ref, z_ref, ax_ref, ay_ref, az_ref, cen_ref,
          cee_ref, cen_a_ref,
          een_ref, eeg_ref, eewr_ref, eews_ref, eewo_ref,
          enn_ref, enga_ref, enge_ref, enwr_ref, enws_ref, enwo_ref,
          out_ref):
    x = x_ref[...]
    y = y_ref[...]
    z = z_ref[...]
    cen = cen_ref[...]
    rbf0 = jnp.exp(-GAMMA * (1e-6 - cen) ** 2)

    jmask = lax.broadcasted_iota(jnp.int32, (NELEC, FEAT), 0) >= NUP

    dx = x[:, :, None] - x[:, None, :]
    dy = y[:, :, None] - y[:, None, :]
    dz = z[:, :, None] - z[:, None, :]
    dee = jnp.sqrt(dx * dx + dy * dy + dz * dz + 1e-12)
    D2 = jnp.concatenate([dee] * NRBF, axis=2)
    R2 = jnp.exp(-GAMMA * (D2 - cee_ref[...].reshape(1, 1, NRBF * NELEC)) ** 2)

    eeg = eeg_ref[...]
    g_up = jnp.where(jmask, eeg[1:2, :], eeg[0:1, :])
    g_dn = jnp.where(jmask, eeg[2:3, :], eeg[1:2, :])

    een = een_ref[...]
    h0 = jnp.where(jmask, een[1:2, :], een[0:1, :])

    rsel = lax.broadcasted_iota(jnp.int32, (NRBF * NELEC, 2 * NRBF), 0)
    csel = lax.broadcasted_iota(jnp.int32, (NRBF * NELEC, 2 * NRBF), 1)
    Sel = jnp.where((rsel // NELEC == csel % NRBF)
                    & ((rsel % NELEC >= NUP) == (csel // NRBF == 1)),
                    1.0, 0.0)
    PQ = lax.dot_general(R2, Sel, (((2,), (0,)), ((), ())),
                         preferred_element_type=jnp.float32)

    a_lo = jnp.concatenate([eeg[0:1] * een[0:1], eeg[1:2] * een[0:1]], 1)
    a_hi = jnp.concatenate([eeg[1:2] * een[1:2], eeg[2:3] * een[1:2]], 1)
    imask2 = lax.broadcasted_iota(jnp.int32, (NELEC, 2 * FEAT), 0) >= NUP
    HG0 = jnp.where(imask2, a_hi, a_lo)

    h = jnp.broadcast_to(h0[None], (BB, NELEC, FEAT))
    delta = None
    for l in range(NLAYERS):
        Wcat = jnp.concatenate([eewr_ref[l], eewr_ref[l]], axis=1)
        rbf0W = jnp.sum(rbf0.reshape(NRBF, 1) * Wcat, axis=0,
                        keepdims=True)
        Wstk = jnp.concatenate([Wcat * a_lo, Wcat * a_hi], axis=0)
        AG = lax.dot_general(PQ, Wstk, (((2,), (0,)), ((), ())),
                             preferred_element_type=jnp.float32)
        AG = AG - (rbf0W * HG0)[None]
        if delta is not None:
            HGd = jnp.concatenate([g_up[None] * delta, g_dn[None] * delta],
                                  axis=2)
            B2 = (HGd[:, None, :, :] * Wcat.reshape(1, NRBF, 1, 2 * FEAT)
                  ).reshape(BB, NRBF * NELEC, 2 * FEAT)
            AG = AG + lax.dot_general(R2, B2, (((2,), (1,)), ((0,), (0,))),
                                      preferred_element_type=jnp.float32)
            AG = AG - rbf0W[None] * HGd
        agg = jnp.concatenate([AG[:, :NUP, :FEAT], AG[:, NUP:, FEAT:]],
                              axis=1)
        upd = jnp.dot(agg.reshape(BB * NELEC, FEAT), eews_ref[l],
                      preferred_element_type=jnp.float32)
        d_new = jnp.tanh(upd).reshape(BB, NELEC, FEAT)
        delta = d_new if delta is None else delta + d_new
        h = h + d_new

    ee_k = jnp.sum(jnp.sum(h, axis=1) * eewo_ref[...], axis=1)

    axr = ax_ref[...].reshape(1, 1, NATOMS)
    ayr = ay_ref[...].reshape(1, 1, NATOMS)
    azr = az_ref[...].reshape(1, 1, NATOMS)
    dxa = x[:, :, None] - axr
    dya = y[:, :, None] - ayr
    dza = z[:, :, None] - azr
    den_ei = jnp.sqrt(dxa * dxa + dya * dya + dza * dza + 1e-12)
    D2e = jnp.concatenate([den_ei] * NRBF, axis=2)
    Ren_e = jnp.exp(
        -GAMMA * (D2e - cen_a_ref[...].reshape(1, 1, NRBF * NATOMS)) ** 2)

    dxb = ax_ref[...].reshape(1, NATOMS, 1) - x[:, None, :]
    dyb = ay_ref[...].reshape(1, NATOMS, 1) - y[:, None, :]
    dzb = az_ref[...].reshape(1, NATOMS, 1) - z[:, None, :]
    den_ai = jnp.sqrt(dxb * dxb + dyb * dyb + dzb * dzb + 1e-12)
    ckk = cen.reshape(1, NRBF, 1, 1)
    Ren_a = jnp.exp(-GAMMA * (den_ai[:, None, :, :] - ckk) ** 2
                    ).reshape(BB, NRBF * NATOMS, NELEC)

    enn = enn_ref[...]
    he = jnp.broadcast_to(jnp.where(jmask, enn[1:2, :], enn[0:1, :])[None],
                          (BB, NELEC, FEAT))
    ha = jnp.broadcast_to(enn[2:2 + NATOMS][None], (BB, NATOMS, FEAT))
    ga = enga_ref[...]
    ge = enge_ref[...]

    for l in range(NLAYERS):
        W = enwr_ref[l]
        gha = ge[None] * ha
        Ben = (gha[:, None, :, :] * W.reshape(1, NRBF, 1, FEAT)
               ).reshape(BB, NRBF * NATOMS, FEAT)
        agg_e = lax.dot_general(Ren_e, Ben, (((2,), (1,)), ((0,), (0,))),
                                preferred_element_type=jnp.float32)
        U = lax.dot_general(Ren_a, he, (((2,), (1,)), ((0,), (0,))),
                            preferred_element_type=jnp.float32)
        Ur = U.reshape(BB, NRBF, NATOMS, FEAT) * W.reshape(1, NRBF, 1, FEAT)
        agg_a = ga[None] * jnp.sum(Ur, axis=1)
        upd_e = jnp.dot(agg_e.reshape(BB * NELEC, FEAT), enws_ref[l],
                        preferred_element_type=jnp.float32)
        upd_a = jnp.dot(agg_a.reshape(BB * NATOMS, FEAT), enws_ref[l],
                        preferred_element_type=jnp.float32)
        he = he + jnp.tanh(upd_e).reshape(BB, NELEC, FEAT)
        ha = ha + jnp.tanh(upd_a).reshape(BB, NATOMS, FEAT)

    nsum = jnp.sum(he, axis=1) + jnp.sum(ha, axis=1)
    en_k = jnp.sum(nsum * enwo_ref[...], axis=1)

    out_ref[...] = jnp.exp(ee_k + en_k).reshape(1, 1, BB)


def kernel(pos, atom_coords, ee_node_emb, ee_edge_emb, ee_Wrbf, ee_Wself,
           ee_Wout, en_node_emb, en_edge_emb, en_Wrbf, en_Wself, en_Wout):
    nb = pos.shape[0]
    xyz = pos.reshape(nb, NELEC, 3)
    X = xyz[:, :, 0]
    Y = xyz[:, :, 1]
    Z = xyz[:, :, 2]
    AX = atom_coords[:, 0].reshape(1, NATOMS)
    AY = atom_coords[:, 1].reshape(1, NATOMS)
    AZ = atom_coords[:, 2].reshape(1, NATOMS)
    cen = jnp.asarray(_CENTERS)
    cee = jnp.asarray(_CENTERS_EE)
    cena = jnp.asarray(_CENTERS_EN)
    eeWoT = ee_Wout.reshape(1, FEAT)
    enWoT = en_Wout.reshape(1, FEAT)
    enGA = en_edge_emb[0::2]
    enGE = en_edge_emb[1::2]

    grid = nb // BB
    full = lambda shape: pl.BlockSpec(shape, lambda i, s=len(shape): (0,) * s)
    out = pl.pallas_call(
        _body,
        grid=(grid,),
        in_specs=[
            pl.BlockSpec((BB, NELEC), lambda i: (i, 0)),
            pl.BlockSpec((BB, NELEC), lambda i: (i, 0)),
            pl.BlockSpec((BB, NELEC), lambda i: (i, 0)),
            full((1, NATOMS)), full((1, NATOMS)), full((1, NATOMS)),
            full((1, NRBF)),
            full((1, NRBF * NELEC)), full((1, NRBF * NATOMS)),
            full((2, FEAT)), full((3, FEAT)),
            full((NLAYERS, NRBF, FEAT)), full((NLAYERS, FEAT, FEAT)),
            full((1, FEAT)),
            full((2 + NATOMS, FEAT)),
            full((NATOMS, FEAT)), full((NATOMS, FEAT)),
            full((NLAYERS, NRBF, FEAT)), full((NLAYERS, FEAT, FEAT)),
            full((1, FEAT)),
        ],
        out_specs=pl.BlockSpec((1, 1, BB), lambda i: (i, 0, 0)),
        out_shape=jax.ShapeDtypeStruct((grid, 1, BB), jnp.float32),
        compiler_params=pltpu.CompilerParams(
            dimension_semantics=("parallel",)),
    )(X, Y, Z, AX, AY, AZ, cen, cee, cena,
      ee_node_emb, ee_edge_emb, ee_Wrbf, ee_Wself, eeWoT,
      en_node_emb, enGA, enGE, en_Wrbf, en_Wself, enWoT)
    return out.reshape(nb, 1)


# speedup vs baseline: 127.2033x; 1.0006x over previous
import numpy as np
import jax
import jax.numpy as jnp
from jax import lax
from jax.experimental import pallas as pl
from jax.experimental.pallas import tpu as pltpu

NELEC = 64
NUP = 32
NATOMS = 16
FEAT = 32
NRBF = 32
NLAYERS = 2
NBATCH = 256
GAMMA = 10.0
BB = 8

_CENTERS = np.linspace(0.0, 10.0, NRBF).astype(np.float32).reshape(1, NRBF)
_CENTERS_EE = np.repeat(_CENTERS.ravel(), NELEC).reshape(1, NRBF * NELEC)
_CENTERS_EN = np.repeat(_CENTERS.ravel(), NATOMS).reshape(1, NRBF * NATOMS)


def _body(x_ref, y_ref, z_ref, ax_ref, ay_ref, az_ref, cen_ref,
          cee_ref, cen_a_ref,
          een_ref, eeg_ref, eewr_ref, eews_ref, eewo_ref,
          enn_ref, enga_ref, enge_ref, enwr_ref, enws_ref, enwo_ref,
          out_ref):
    x = x_ref[...]
    y = y_ref[...]
    z = z_ref[...]
    cen = cen_ref[...]
    rbf0 = jnp.exp(-GAMMA * (1e-6 - cen) ** 2)

    jmask = lax.broadcasted_iota(jnp.int32, (NELEC, FEAT), 0) >= NUP

    dx = x[:, :, None] - x[:, None, :]
    dy = y[:, :, None] - y[:, None, :]
    dz = z[:, :, None] - z[:, None, :]
    dee = jnp.sqrt(dx * dx + dy * dy + dz * dz + 1e-12)
    D2 = jnp.concatenate([dee] * NRBF, axis=2)
    R2 = jnp.exp(-GAMMA * (D2 - cee_ref[...].reshape(1, 1, NRBF * NELEC)) ** 2)

    eeg = eeg_ref[...]
    g_up = jnp.where(jmask, eeg[1:2, :], eeg[0:1, :])
    g_dn = jnp.where(jmask, eeg[2:3, :], eeg[1:2, :])

    een = een_ref[...]
    h0 = jnp.where(jmask, een[1:2, :], een[0:1, :])

    h = jnp.broadcast_to(h0[None], (BB, NELEC, FEAT))
    for l in range(NLAYERS):
        Wcat = jnp.concatenate([eewr_ref[l], eewr_ref[l]], axis=1)
        rbf0W = jnp.sum(rbf0.reshape(NRBF, 1) * Wcat, axis=0,
                        keepdims=True)
        if l == 0:
            HG0 = jnp.concatenate([g_up * h0, g_dn * h0], axis=1)
            B2 = (HG0[None, :, :] * Wcat[:, None, :]
                  ).reshape(NRBF * NELEC, 2 * FEAT)
            AG = lax.dot_general(R2, B2, (((2,), (0,)), ((), ())),
                                 preferred_element_type=jnp.float32)
            AG = AG - (rbf0W * HG0)[None]
        else:
            HG = jnp.concatenate([g_up[None] * h, g_dn[None] * h], axis=2)
            B2 = (HG[:, None, :, :] * Wcat.reshape(1, NRBF, 1, 2 * FEAT)
                  ).reshape(BB, NRBF * NELEC, 2 * FEAT)
            AG = lax.dot_general(R2, B2, (((2,), (1,)), ((0,), (0,))),
                                 preferred_element_type=jnp.float32)
            AG = AG - rbf0W[None] * HG
        agg = jnp.concatenate([AG[:, :NUP, :FEAT], AG[:, NUP:, FEAT:]],
                              axis=1)
        upd = jnp.dot(agg.reshape(BB * NELEC, FEAT), eews_ref[l],
                      preferred_element_type=jnp.float32)
        h = h + jnp.tanh(upd).reshape(BB, NELEC, FEAT)

    ee_k = jnp.sum(jnp.sum(h, axis=1) * eewo_ref[...], axis=1)

    axr = ax_ref[...].reshape(1, 1, NATOMS)
    ayr = ay_ref[...].reshape(1, 1, NATOMS)
    azr = az_ref[...].reshape(1, 1, NATOMS)
    dxa = x[:, :, None] - axr
    dya = y[:, :, None] - ayr
    dza = z[:, :, None] - azr
    den_ei = jnp.sqrt(dxa * dxa + dya * dya + dza * dza + 1e-12)
    D2e = jnp.concatenate([den_ei] * NRBF, axis=2)
    Ren_e = jnp.exp(
        -GAMMA * (D2e - cen_a_ref[...].reshape(1, 1, NRBF * NATOMS)) ** 2)

    dxb = ax_ref[...].reshape(1, NATOMS, 1) - x[:, None, :]
    dyb = ay_ref[...].reshape(1, NATOMS, 1) - y[:, None, :]
    dzb = az_ref[...].reshape(1, NATOMS, 1) - z[:, None, :]
    den_ai = jnp.sqrt(dxb * dxb + dyb * dyb + dzb * dzb + 1e-12)
    ckk = cen.reshape(1, NRBF, 1, 1)
    Ren_a = jnp.exp(-GAMMA * (den_ai[:, None, :, :] - ckk) ** 2
                    ).reshape(BB, NRBF * NATOMS, NELEC)

    enn = enn_ref[...]
    he0 = jnp.where(jmask, enn[1:2, :], enn[0:1, :])
    ha0 = enn[2:2 + NATOMS]
    he = jnp.broadcast_to(he0[None], (BB, NELEC, FEAT))
    ha = jnp.broadcast_to(ha0[None], (BB, NATOMS, FEAT))
    ga = enga_ref[...]
    ge = enge_ref[...]
    for l in range(NLAYERS):
        W = enwr_ref[l]
        if l == 0:
            Ben = (ge * ha0)[None, :, :] * W[:, None, :]
            agg_e = lax.dot_general(
                Ren_e, Ben.reshape(NRBF * NATOMS, FEAT),
                (((2,), (0,)), ((), ())),
                preferred_element_type=jnp.float32)
            U = lax.dot_general(Ren_a, he0, (((2,), (0,)), ((), ())),
                                preferred_element_type=jnp.float32)
        else:
            gha = ge[None] * ha
            Ben = (gha[:, None, :, :] * W.reshape(1, NRBF, 1, FEAT)
                   ).reshape(BB, NRBF * NATOMS, FEAT)
            agg_e = lax.dot_general(Ren_e, Ben, (((2,), (1,)), ((0,), (0,))),
                                    preferred_element_type=jnp.float32)
            U = lax.dot_general(Ren_a, he, (((2,), (1,)), ((0,), (0,))),
                                preferred_element_type=jnp.float32)
        Ur = U.reshape(BB, NRBF, NATOMS, FEAT) * W.reshape(1, NRBF, 1, FEAT)
        agg_a = ga[None] * jnp.sum(Ur, axis=1)
        upd_e = jnp.dot(agg_e.reshape(BB * NELEC, FEAT), enws_ref[l],
                        preferred_element_type=jnp.float32)
        upd_a = jnp.dot(agg_a.reshape(BB * NATOMS, FEAT), enws_ref[l],
                        preferred_element_type=jnp.float32)
        he = he + jnp.tanh(upd_e).reshape(BB, NELEC, FEAT)
        ha = ha + jnp.tanh(upd_a).reshape(BB, NATOMS, FEAT)

    nsum = jnp.sum(he, axis=1) + jnp.sum(ha, axis=1)
    en_k = jnp.sum(nsum * enwo_ref[...], axis=1)

    out_ref[...] = jnp.exp(ee_k + en_k).reshape(1, 1, BB)


def kernel(pos, atom_coords, ee_node_emb, ee_edge_emb, ee_Wrbf, ee_Wself,
           ee_Wout, en_node_emb, en_edge_emb, en_Wrbf, en_Wself, en_Wout):
    nb = pos.shape[0]
    xyz = pos.reshape(nb, NELEC, 3)
    X = xyz[:, :, 0]
    Y = xyz[:, :, 1]
    Z = xyz[:, :, 2]
    AX = atom_coords[:, 0].reshape(1, NATOMS)
    AY = atom_coords[:, 1].reshape(1, NATOMS)
    AZ = atom_coords[:, 2].reshape(1, NATOMS)
    cen = jnp.asarray(_CENTERS)
    cee = jnp.asarray(_CENTERS_EE)
    cena = jnp.asarray(_CENTERS_EN)
    eeWoT = ee_Wout.reshape(1, FEAT)
    enWoT = en_Wout.reshape(1, FEAT)
    enGA = en_edge_emb[0::2]
    enGE = en_edge_emb[1::2]

    grid = nb // BB
    full = lambda shape: pl.BlockSpec(shape, lambda i, s=len(shape): (0,) * s)
    out = pl.pallas_call(
        _body,
        grid=(grid,),
        in_specs=[
            pl.BlockSpec((BB, NELEC), lambda i: (i, 0)),
            pl.BlockSpec((BB, NELEC), lambda i: (i, 0)),
            pl.BlockSpec((BB, NELEC), lambda i: (i, 0)),
            full((1, NATOMS)), full((1, NATOMS)), full((1, NATOMS)),
            full((1, NRBF)),
            full((1, NRBF * NELEC)), full((1, NRBF * NATOMS)),
            full((2, FEAT)), full((3, FEAT)),
            full((NLAYERS, NRBF, FEAT)), full((NLAYERS, FEAT, FEAT)),
            full((1, FEAT)),
            full((2 + NATOMS, FEAT)),
            full((NATOMS, FEAT)), full((NATOMS, FEAT)),
            full((NLAYERS, NRBF, FEAT)), full((NLAYERS, FEAT, FEAT)),
            full((1, FEAT)),
        ],
        out_specs=pl.BlockSpec((1, 1, BB), lambda i: (i, 0, 0)),
        out_shape=jax.ShapeDtypeStruct((grid, 1, BB), jnp.float32),
        compiler_params=pltpu.CompilerParams(
            dimension_semantics=("parallel",)),
    )(X, Y, Z, AX, AY, AZ, cen, cee, cena,
      ee_node_emb, ee_edge_emb, ee_Wrbf, ee_Wself, eeWoT,
      en_node_emb, enGA, enGE, en_Wrbf, en_Wself, enWoT)
    return out.reshape(nb, 1)


# speedup vs baseline: 144.2962x; 1.1344x over previous
import numpy as np
import jax
import jax.numpy as jnp
from jax import lax
from jax.experimental import pallas as pl
from jax.experimental.pallas import tpu as pltpu

NELEC = 64
NUP = 32
NATOMS = 16
FEAT = 32
NRBF = 32
NLAYERS = 2
NBATCH = 256
GAMMA = 10.0
BB = 16

_CENTERS = np.linspace(0.0, 10.0, NRBF).astype(np.float32).reshape(1, NRBF)
_CENTERS_EE = np.repeat(_CENTERS.ravel(), NELEC).reshape(1, NRBF * NELEC)
_CENTERS_EN = np.repeat(_CENTERS.ravel(), NATOMS).reshape(1, NRBF * NATOMS)


def _body(x_ref, y_ref, z_ref, ax_ref, ay_ref, az_ref, cen_ref,
          cee_ref, cen_a_ref,
          een_ref, eeg_ref, eewr_ref, eews_ref, eewo_ref,
          enn_ref, enga_ref, enge_ref, enwr_ref, enws_ref, enwo_ref,
          out_ref):
    x = x_ref[...]
    y = y_ref[...]
    z = z_ref[...]
    cen = cen_ref[...]
    rbf0 = jnp.exp(-GAMMA * (1e-6 - cen) ** 2)

    jmask = lax.broadcasted_iota(jnp.int32, (NELEC, FEAT), 0) >= NUP

    dx = x[:, :, None] - x[:, None, :]
    dy = y[:, :, None] - y[:, None, :]
    dz = z[:, :, None] - z[:, None, :]
    dee = jnp.sqrt(dx * dx + dy * dy + dz * dz + 1e-12)
    D2 = jnp.concatenate([dee] * NRBF, axis=2)
    R2 = jnp.exp(-GAMMA * (D2 - cee_ref[...].reshape(1, 1, NRBF * NELEC)) ** 2)

    eeg = eeg_ref[...]
    g_up = jnp.where(jmask, eeg[1:2, :], eeg[0:1, :])
    g_dn = jnp.where(jmask, eeg[2:3, :], eeg[1:2, :])

    een = een_ref[...]
    h0 = jnp.where(jmask, een[1:2, :], een[0:1, :])

    h = jnp.broadcast_to(h0[None], (BB, NELEC, FEAT))
    for l in range(NLAYERS):
        Wcat = jnp.concatenate([eewr_ref[l], eewr_ref[l]], axis=1)
        rbf0W = jnp.sum(rbf0.reshape(NRBF, 1) * Wcat, axis=0,
                        keepdims=True)
        if l == 0:
            HG0 = jnp.concatenate([g_up * h0, g_dn * h0], axis=1)
            B2 = (HG0[None, :, :] * Wcat[:, None, :]
                  ).reshape(NRBF * NELEC, 2 * FEAT)
            AG = lax.dot_general(R2, B2, (((2,), (0,)), ((), ())),
                                 preferred_element_type=jnp.float32)
            AG = AG - (rbf0W * HG0)[None]
        else:
            HG = jnp.concatenate([g_up[None] * h, g_dn[None] * h], axis=2)
            B2 = (HG[:, None, :, :] * Wcat.reshape(1, NRBF, 1, 2 * FEAT)
                  ).reshape(BB, NRBF * NELEC, 2 * FEAT)
            AG = lax.dot_general(R2, B2, (((2,), (1,)), ((0,), (0,))),
                                 preferred_element_type=jnp.float32)
            AG = AG - rbf0W[None] * HG
        agg = jnp.concatenate([AG[:, :NUP, :FEAT], AG[:, NUP:, FEAT:]],
                              axis=1)
        upd = jnp.dot(agg.reshape(BB * NELEC, FEAT), eews_ref[l],
                      preferred_element_type=jnp.float32)
        h = h + jnp.tanh(upd).reshape(BB, NELEC, FEAT)

    ee_k = jnp.sum(jnp.sum(h, axis=1) * eewo_ref[...], axis=1)

    axr = ax_ref[...].reshape(1, 1, NATOMS)
    ayr = ay_ref[...].reshape(1, 1, NATOMS)
    azr = az_ref[...].reshape(1, 1, NATOMS)
    dxa = x[:, :, None] - axr
    dya = y[:, :, None] - ayr
    dza = z[:, :, None] - azr
    den_ei = jnp.sqrt(dxa * dxa + dya * dya + dza * dza + 1e-12)
    D2e = jnp.concatenate([den_ei] * NRBF, axis=2)
    Ren_e = jnp.exp(
        -GAMMA * (D2e - cen_a_ref[...].reshape(1, 1, NRBF * NATOMS)) ** 2)

    dxb = ax_ref[...].reshape(1, NATOMS, 1) - x[:, None, :]
    dyb = ay_ref[...].reshape(1, NATOMS, 1) - y[:, None, :]
    dzb = az_ref[...].reshape(1, NATOMS, 1) - z[:, None, :]
    den_ai = jnp.sqrt(dxb * dxb + dyb * dyb + dzb * dzb + 1e-12)
    ckk = cen.reshape(1, NRBF, 1, 1)
    Ren_a = jnp.exp(-GAMMA * (den_ai[:, None, :, :] - ckk) ** 2
                    ).reshape(BB, NRBF * NATOMS, NELEC)

    enn = enn_ref[...]
    he0 = jnp.where(jmask, enn[1:2, :], enn[0:1, :])
    ha0 = enn[2:2 + NATOMS]
    he = jnp.broadcast_to(he0[None], (BB, NELEC, FEAT))
    ha = jnp.broadcast_to(ha0[None], (BB, NATOMS, FEAT))
    ga = enga_ref[...]
    ge = enge_ref[...]
    for l in range(NLAYERS):
        W = enwr_ref[l]
        if l == 0:
            Ben = (ge * ha0)[None, :, :] * W[:, None, :]
            agg_e = lax.dot_general(
                Ren_e, Ben.reshape(NRBF * NATOMS, FEAT),
                (((2,), (0,)), ((), ())),
                preferred_element_type=jnp.float32)
            U = lax.dot_general(Ren_a, he0, (((2,), (0,)), ((), ())),
                                preferred_element_type=jnp.float32)
        else:
            gha = ge[None] * ha
            Ben = (gha[:, None, :, :] * W.reshape(1, NRBF, 1, FEAT)
                   ).reshape(BB, NRBF * NATOMS, FEAT)
            agg_e = lax.dot_general(Ren_e, Ben, (((2,), (1,)), ((0,), (0,))),
                                    preferred_element_type=jnp.float32)
            U = lax.dot_general(Ren_a, he, (((2,), (1,)), ((0,), (0,))),
                                preferred_element_type=jnp.float32)
        Ur = U.reshape(BB, NRBF, NATOMS, FEAT) * W.reshape(1, NRBF, 1, FEAT)
        agg_a = ga[None] * jnp.sum(Ur, axis=1)
        upd_e = jnp.dot(agg_e.reshape(BB * NELEC, FEAT), enws_ref[l],
                        preferred_element_type=jnp.float32)
        upd_a = jnp.dot(agg_a.reshape(BB * NATOMS, FEAT), enws_ref[l],
                        preferred_element_type=jnp.float32)
        he = he + jnp.tanh(upd_e).reshape(BB, NELEC, FEAT)
        ha = ha + jnp.tanh(upd_a).reshape(BB, NATOMS, FEAT)

    nsum = jnp.sum(he, axis=1) + jnp.sum(ha, axis=1)
    en_k = jnp.sum(nsum * enwo_ref[...], axis=1)

    out_ref[...] = jnp.exp(ee_k + en_k).reshape(1, 1, BB)


def kernel(pos, atom_coords, ee_node_emb, ee_edge_emb, ee_Wrbf, ee_Wself,
           ee_Wout, en_node_emb, en_edge_emb, en_Wrbf, en_Wself, en_Wout):
    nb = pos.shape[0]
    xyz = pos.reshape(nb, NELEC, 3)
    X = xyz[:, :, 0]
    Y = xyz[:, :, 1]
    Z = xyz[:, :, 2]
    AX = atom_coords[:, 0].reshape(1, NATOMS)
    AY = atom_coords[:, 1].reshape(1, NATOMS)
    AZ = atom_coords[:, 2].reshape(1, NATOMS)
    cen = jnp.asarray(_CENTERS)
    cee = jnp.asarray(_CENTERS_EE)
    cena = jnp.asarray(_CENTERS_EN)
    eeWoT = ee_Wout.reshape(1, FEAT)
    enWoT = en_Wout.reshape(1, FEAT)
    enGA = en_edge_emb[0::2]
    enGE = en_edge_emb[1::2]

    grid = nb // BB
    full = lambda shape: pl.BlockSpec(shape, lambda i, s=len(shape): (0,) * s)
    out = pl.pallas_call(
        _body,
        grid=(grid,),
        in_specs=[
            pl.BlockSpec((BB, NELEC), lambda i: (i, 0)),
            pl.BlockSpec((BB, NELEC), lambda i: (i, 0)),
            pl.BlockSpec((BB, NELEC), lambda i: (i, 0)),
            full((1, NATOMS)), full((1, NATOMS)), full((1, NATOMS)),
            full((1, NRBF)),
            full((1, NRBF * NELEC)), full((1, NRBF * NATOMS)),
            full((2, FEAT)), full((3, FEAT)),
            full((NLAYERS, NRBF, FEAT)), full((NLAYERS, FEAT, FEAT)),
            full((1, FEAT)),
            full((2 + NATOMS, FEAT)),
            full((NATOMS, FEAT)), full((NATOMS, FEAT)),
            full((NLAYERS, NRBF, FEAT)), full((NLAYERS, FEAT, FEAT)),
            full((1, FEAT)),
        ],
        out_specs=pl.BlockSpec((1, 1, BB), lambda i: (i, 0, 0)),
        out_shape=jax.ShapeDtypeStruct((grid, 1, BB), jnp.float32),
        compiler_params=pltpu.CompilerParams(
            dimension_semantics=("parallel",)),
    )(X, Y, Z, AX, AY, AZ, cen, cee, cena,
      ee_node_emb, ee_edge_emb, ee_Wrbf, ee_Wself, eeWoT,
      en_node_emb, enGA, enGE, en_Wrbf, en_Wself, enWoT)
    return out.reshape(nb, 1)


# speedup vs baseline: 148.4825x; 1.0290x over previous
import numpy as np
import jax
import jax.numpy as jnp
from jax import lax
from jax.experimental import pallas as pl
from jax.experimental.pallas import tpu as pltpu

NELEC = 64
NUP = 32
NATOMS = 16
FEAT = 32
NRBF = 32
NLAYERS = 2
NBATCH = 256
GAMMA = 10.0
BB = 32

_CENTERS = np.linspace(0.0, 10.0, NRBF).astype(np.float32).reshape(1, NRBF)
_CENTERS_EE = np.repeat(_CENTERS.ravel(), NELEC).reshape(1, NRBF * NELEC)
_CENTERS_EN = np.repeat(_CENTERS.ravel(), NATOMS).reshape(1, NRBF * NATOMS)


def _body(x_ref, y_ref, z_ref, ax_ref, ay_ref, az_ref, cen_ref,
          cee_ref, cen_a_ref,
          een_ref, eeg_ref, eewr_ref, eews_ref, eewo_ref,
          enn_ref, enga_ref, enge_ref, enwr_ref, enws_ref, enwo_ref,
          out_ref):
    x = x_ref[...]
    y = y_ref[...]
    z = z_ref[...]
    cen = cen_ref[...]
    rbf0 = jnp.exp(-GAMMA * (1e-6 - cen) ** 2)

    jmask = lax.broadcasted_iota(jnp.int32, (NELEC, FEAT), 0) >= NUP

    dx = x[:, :, None] - x[:, None, :]
    dy = y[:, :, None] - y[:, None, :]
    dz = z[:, :, None] - z[:, None, :]
    dee = jnp.sqrt(dx * dx + dy * dy + dz * dz + 1e-12)
    D2 = jnp.concatenate([dee] * NRBF, axis=2)
    R2 = jnp.exp(-GAMMA * (D2 - cee_ref[...].reshape(1, 1, NRBF * NELEC)) ** 2)

    eeg = eeg_ref[...]
    g_up = jnp.where(jmask, eeg[1:2, :], eeg[0:1, :])
    g_dn = jnp.where(jmask, eeg[2:3, :], eeg[1:2, :])

    een = een_ref[...]
    h0 = jnp.where(jmask, een[1:2, :], een[0:1, :])

    h = jnp.broadcast_to(h0[None], (BB, NELEC, FEAT))
    for l in range(NLAYERS):
        Wcat = jnp.concatenate([eewr_ref[l], eewr_ref[l]], axis=1)
        rbf0W = jnp.sum(rbf0.reshape(NRBF, 1) * Wcat, axis=0,
                        keepdims=True)
        if l == 0:
            HG0 = jnp.concatenate([g_up * h0, g_dn * h0], axis=1)
            B2 = (HG0[None, :, :] * Wcat[:, None, :]
                  ).reshape(NRBF * NELEC, 2 * FEAT)
            AG = lax.dot_general(R2, B2, (((2,), (0,)), ((), ())),
                                 preferred_element_type=jnp.float32)
            AG = AG - (rbf0W * HG0)[None]
        else:
            HG = jnp.concatenate([g_up[None] * h, g_dn[None] * h], axis=2)
            B2 = (HG[:, None, :, :] * Wcat.reshape(1, NRBF, 1, 2 * FEAT)
                  ).reshape(BB, NRBF * NELEC, 2 * FEAT)
            AG = lax.dot_general(R2, B2, (((2,), (1,)), ((0,), (0,))),
                                 preferred_element_type=jnp.float32)
            AG = AG - rbf0W[None] * HG
        agg = jnp.concatenate([AG[:, :NUP, :FEAT], AG[:, NUP:, FEAT:]],
                              axis=1)
        upd = jnp.dot(agg.reshape(BB * NELEC, FEAT), eews_ref[l],
                      preferred_element_type=jnp.float32)
        h = h + jnp.tanh(upd).reshape(BB, NELEC, FEAT)

    ee_k = jnp.sum(jnp.sum(h, axis=1) * eewo_ref[...], axis=1)

    axr = ax_ref[...].reshape(1, 1, NATOMS)
    ayr = ay_ref[...].reshape(1, 1, NATOMS)
    azr = az_ref[...].reshape(1, 1, NATOMS)
    dxa = x[:, :, None] - axr
    dya = y[:, :, None] - ayr
    dza = z[:, :, None] - azr
    den_ei = jnp.sqrt(dxa * dxa + dya * dya + dza * dza + 1e-12)
    D2e = jnp.concatenate([den_ei] * NRBF, axis=2)
    Ren_e = jnp.exp(
        -GAMMA * (D2e - cen_a_ref[...].reshape(1, 1, NRBF * NATOMS)) ** 2)

    dxb = ax_ref[...].reshape(1, NATOMS, 1) - x[:, None, :]
    dyb = ay_ref[...].reshape(1, NATOMS, 1) - y[:, None, :]
    dzb = az_ref[...].reshape(1, NATOMS, 1) - z[:, None, :]
    den_ai = jnp.sqrt(dxb * dxb + dyb * dyb + dzb * dzb + 1e-12)
    ckk = cen.reshape(1, NRBF, 1, 1)
    Ren_a = jnp.exp(-GAMMA * (den_ai[:, None, :, :] - ckk) ** 2
                    ).reshape(BB, NRBF * NATOMS, NELEC)

    enn = enn_ref[...]
    he0 = jnp.where(jmask, enn[1:2, :], enn[0:1, :])
    ha0 = enn[2:2 + NATOMS]
    he = jnp.broadcast_to(he0[None], (BB, NELEC, FEAT))
    ha = jnp.broadcast_to(ha0[None], (BB, NATOMS, FEAT))
    ga = enga_ref[...]
    ge = enge_ref[...]
    for l in range(NLAYERS):
        W = enwr_ref[l]
        if l == 0:
            Ben = (ge * ha0)[None, :, :] * W[:, None, :]
            agg_e = lax.dot_general(
                Ren_e, Ben.reshape(NRBF * NATOMS, FEAT),
                (((2,), (0,)), ((), ())),
                preferred_element_type=jnp.float32)
            U = lax.dot_general(Ren_a, he0, (((2,), (0,)), ((), ())),
                                preferred_element_type=jnp.float32)
        else:
            gha = ge[None] * ha
            Ben = (gha[:, None, :, :] * W.reshape(1, NRBF, 1, FEAT)
                   ).reshape(BB, NRBF * NATOMS, FEAT)
            agg_e = lax.dot_general(Ren_e, Ben, (((2,), (1,)), ((0,), (0,))),
                                    preferred_element_type=jnp.float32)
            U = lax.dot_general(Ren_a, he, (((2,), (1,)), ((0,), (0,))),
                                preferred_element_type=jnp.float32)
        Ur = U.reshape(BB, NRBF, NATOMS, FEAT) * W.reshape(1, NRBF, 1, FEAT)
        agg_a = ga[None] * jnp.sum(Ur, axis=1)
        upd_e = jnp.dot(agg_e.reshape(BB * NELEC, FEAT), enws_ref[l],
                        preferred_element_type=jnp.float32)
        upd_a = jnp.dot(agg_a.reshape(BB * NATOMS, FEAT), enws_ref[l],
                        preferred_element_type=jnp.float32)
        he = he + jnp.tanh(upd_e).reshape(BB, NELEC, FEAT)
        ha = ha + jnp.tanh(upd_a).reshape(BB, NATOMS, FEAT)

    nsum = jnp.sum(he, axis=1) + jnp.sum(ha, axis=1)
    en_k = jnp.sum(nsum * enwo_ref[...], axis=1)

    out_ref[...] = jnp.exp(ee_k + en_k).reshape(1, 1, BB)


def kernel(pos, atom_coords, ee_node_emb, ee_edge_emb, ee_Wrbf, ee_Wself,
           ee_Wout, en_node_emb, en_edge_emb, en_Wrbf, en_Wself, en_Wout):
    nb = pos.shape[0]
    xyz = pos.reshape(nb, NELEC, 3)
    X = xyz[:, :, 0]
    Y = xyz[:, :, 1]
    Z = xyz[:, :, 2]
    AX = atom_coords[:, 0].reshape(1, NATOMS)
    AY = atom_coords[:, 1].reshape(1, NATOMS)
    AZ = atom_coords[:, 2].reshape(1, NATOMS)
    cen = jnp.asarray(_CENTERS)
    cee = jnp.asarray(_CENTERS_EE)
    cena = jnp.asarray(_CENTERS_EN)
    eeWoT = ee_Wout.reshape(1, FEAT)
    enWoT = en_Wout.reshape(1, FEAT)
    enGA = en_edge_emb[0::2]
    enGE = en_edge_emb[1::2]

    grid = nb // BB
    full = lambda shape: pl.BlockSpec(shape, lambda i, s=len(shape): (0,) * s)
    out = pl.pallas_call(
        _body,
        grid=(grid,),
        in_specs=[
            pl.BlockSpec((BB, NELEC), lambda i: (i, 0)),
            pl.BlockSpec((BB, NELEC), lambda i: (i, 0)),
            pl.BlockSpec((BB, NELEC), lambda i: (i, 0)),
            full((1, NATOMS)), full((1, NATOMS)), full((1, NATOMS)),
            full((1, NRBF)),
            full((1, NRBF * NELEC)), full((1, NRBF * NATOMS)),
            full((2, FEAT)), full((3, FEAT)),
            full((NLAYERS, NRBF, FEAT)), full((NLAYERS, FEAT, FEAT)),
            full((1, FEAT)),
            full((2 + NATOMS, FEAT)),
            full((NATOMS, FEAT)), full((NATOMS, FEAT)),
            full((NLAYERS, NRBF, FEAT)), full((NLAYERS, FEAT, FEAT)),
            full((1, FEAT)),
        ],
        out_specs=pl.BlockSpec((1, 1, BB), lambda i: (i, 0, 0)),
        out_shape=jax.ShapeDtypeStruct((grid, 1, BB), jnp.float32),
        compiler_params=pltpu.CompilerParams(
            dimension_semantics=("parallel",)),
    )(X, Y, Z, AX, AY, AZ, cen, cee, cena,
      ee_node_emb, ee_edge_emb, ee_Wrbf, ee_Wself, eeWoT,
      en_node_emb, enGA, enGE, en_Wrbf, en_Wself, enWoT)
    return out.reshape(nb, 1)


# speedup vs baseline: 149.5074x; 1.0069x over previous
import numpy as np
import jax
import jax.numpy as jnp
from jax import lax
from jax.experimental import pallas as pl
from jax.experimental.pallas import tpu as pltpu

NELEC = 64
NUP = 32
NATOMS = 16
FEAT = 32
NRBF = 32
NLAYERS = 2
NBATCH = 256
GAMMA = 10.0
BB = 32

_CENTERS = np.linspace(0.0, 10.0, NRBF).astype(np.float32).reshape(1, NRBF)
_CENTERS_EE = np.repeat(_CENTERS.ravel(), NELEC).reshape(1, NRBF * NELEC)
_CENTERS_EN = np.repeat(_CENTERS.ravel(), NATOMS).reshape(1, NRBF * NATOMS)


def _body(x_ref, y_ref, z_ref, ax_ref, ay_ref, az_ref, cen_ref,
          cee_ref, cen_a_ref,
          een_ref, eeg_ref, eewr_ref, eews_ref, eewo_ref,
          enn_ref, enga_ref, enge_ref, enwr_ref, enws_ref, enwo_ref,
          out_ref):
    x = x_ref[...]
    y = y_ref[...]
    z = z_ref[...]
    cen = cen_ref[...]
    rbf0 = jnp.exp(-GAMMA * (1e-6 - cen) ** 2)

    jmask = lax.broadcasted_iota(jnp.int32, (NELEC, FEAT), 0) >= NUP

    dx = x[:, :, None] - x[:, None, :]
    dy = y[:, :, None] - y[:, None, :]
    dz = z[:, :, None] - z[:, None, :]
    dee = jnp.sqrt(dx * dx + dy * dy + dz * dz + 1e-12)
    D2 = jnp.concatenate([dee] * NRBF, axis=2)
    R2 = jnp.exp(-GAMMA * (D2 - cee_ref[...].reshape(1, 1, NRBF * NELEC)) ** 2
                 ).astype(jnp.bfloat16)

    eeg = eeg_ref[...]
    g_up = jnp.where(jmask, eeg[1:2, :], eeg[0:1, :])
    g_dn = jnp.where(jmask, eeg[2:3, :], eeg[1:2, :])

    een = een_ref[...]
    h0 = jnp.where(jmask, een[1:2, :], een[0:1, :])

    h = jnp.broadcast_to(h0[None], (BB, NELEC, FEAT))
    for l in range(NLAYERS):
        Wcat = jnp.concatenate([eewr_ref[l], eewr_ref[l]], axis=1)
        rbf0W = jnp.sum(rbf0.reshape(NRBF, 1) * Wcat, axis=0,
                        keepdims=True)
        if l == 0:
            HG0 = jnp.concatenate([g_up * h0, g_dn * h0], axis=1)
            B2 = (HG0[None, :, :] * Wcat[:, None, :]
                  ).reshape(NRBF * NELEC, 2 * FEAT).astype(jnp.bfloat16)
            AG = lax.dot_general(R2, B2, (((2,), (0,)), ((), ())),
                                 preferred_element_type=jnp.float32)
            AG = AG - (rbf0W * HG0)[None]
        else:
            HG = jnp.concatenate([g_up[None] * h, g_dn[None] * h], axis=2)
            B2 = (HG[:, None, :, :] * Wcat.reshape(1, NRBF, 1, 2 * FEAT)
                  ).reshape(BB, NRBF * NELEC, 2 * FEAT).astype(jnp.bfloat16)
            AG = lax.dot_general(R2, B2, (((2,), (1,)), ((0,), (0,))),
                                 preferred_element_type=jnp.float32)
            AG = AG - rbf0W[None] * HG
        agg = jnp.concatenate([AG[:, :NUP, :FEAT], AG[:, NUP:, FEAT:]],
                              axis=1)
        upd = jnp.dot(agg.reshape(BB * NELEC, FEAT), eews_ref[l],
                      preferred_element_type=jnp.float32)
        h = h + jnp.tanh(upd).reshape(BB, NELEC, FEAT)

    ee_k = jnp.sum(jnp.sum(h, axis=1) * eewo_ref[...], axis=1)

    axr = ax_ref[...].reshape(1, 1, NATOMS)
    ayr = ay_ref[...].reshape(1, 1, NATOMS)
    azr = az_ref[...].reshape(1, 1, NATOMS)
    dxa = x[:, :, None] - axr
    dya = y[:, :, None] - ayr
    dza = z[:, :, None] - azr
    den_ei = jnp.sqrt(dxa * dxa + dya * dya + dza * dza + 1e-12)
    D2e = jnp.concatenate([den_ei] * NRBF, axis=2)
    Ren_e = jnp.exp(
        -GAMMA * (D2e - cen_a_ref[...].reshape(1, 1, NRBF * NATOMS)) ** 2
    ).astype(jnp.bfloat16)

    dxb = ax_ref[...].reshape(1, NATOMS, 1) - x[:, None, :]
    dyb = ay_ref[...].reshape(1, NATOMS, 1) - y[:, None, :]
    dzb = az_ref[...].reshape(1, NATOMS, 1) - z[:, None, :]
    den_ai = jnp.sqrt(dxb * dxb + dyb * dyb + dzb * dzb + 1e-12)
    ckk = cen.reshape(1, NRBF, 1, 1)
    Ren_a = jnp.exp(-GAMMA * (den_ai[:, None, :, :] - ckk) ** 2
                    ).reshape(BB, NRBF * NATOMS, NELEC
                              ).astype(jnp.bfloat16)

    enn = enn_ref[...]
    he0 = jnp.where(jmask, enn[1:2, :], enn[0:1, :])
    ha0 = enn[2:2 + NATOMS]
    he = jnp.broadcast_to(he0[None], (BB, NELEC, FEAT))
    ha = jnp.broadcast_to(ha0[None], (BB, NATOMS, FEAT))
    ga = enga_ref[...]
    ge = enge_ref[...]
    for l in range(NLAYERS):
        W = enwr_ref[l]
        if l == 0:
            Ben = (ge * ha0)[None, :, :] * W[:, None, :]
            agg_e = lax.dot_general(
                Ren_e, Ben.reshape(NRBF * NATOMS, FEAT).astype(jnp.bfloat16),
                (((2,), (0,)), ((), ())),
                preferred_element_type=jnp.float32)
            U = lax.dot_general(Ren_a, he0.astype(jnp.bfloat16),
                                (((2,), (0,)), ((), ())),
                                preferred_element_type=jnp.float32)
        else:
            gha = ge[None] * ha
            Ben = (gha[:, None, :, :] * W.reshape(1, NRBF, 1, FEAT)
                   ).reshape(BB, NRBF * NATOMS, FEAT).astype(jnp.bfloat16)
            agg_e = lax.dot_general(Ren_e, Ben, (((2,), (1,)), ((0,), (0,))),
                                    preferred_element_type=jnp.float32)
            U = lax.dot_general(Ren_a, he.astype(jnp.bfloat16),
                                (((2,), (1,)), ((0,), (0,))),
                                preferred_element_type=jnp.float32)
        Ur = U.reshape(BB, NRBF, NATOMS, FEAT) * W.reshape(1, NRBF, 1, FEAT)
        agg_a = ga[None] * jnp.sum(Ur, axis=1)
        upd_e = jnp.dot(agg_e.reshape(BB * NELEC, FEAT), enws_ref[l],
                        preferred_element_type=jnp.float32)
        upd_a = jnp.dot(agg_a.reshape(BB * NATOMS, FEAT), enws_ref[l],
                        preferred_element_type=jnp.float32)
        he = he + jnp.tanh(upd_e).reshape(BB, NELEC, FEAT)
        ha = ha + jnp.tanh(upd_a).reshape(BB, NATOMS, FEAT)

    nsum = jnp.sum(he, axis=1) + jnp.sum(ha, axis=1)
    en_k = jnp.sum(nsum * enwo_ref[...], axis=1)

    out_ref[...] = jnp.exp(ee_k + en_k).reshape(1, 1, BB)


def kernel(pos, atom_coords, ee_node_emb, ee_edge_emb, ee_Wrbf, ee_Wself,
           ee_Wout, en_node_emb, en_edge_emb, en_Wrbf, en_Wself, en_Wout):
    nb = pos.shape[0]
    xyz = pos.reshape(nb, NELEC, 3)
    X = xyz[:, :, 0]
    Y = xyz[:, :, 1]
    Z = xyz[:, :, 2]
    AX = atom_coords[:, 0].reshape(1, NATOMS)
    AY = atom_coords[:, 1].reshape(1, NATOMS)
    AZ = atom_coords[:, 2].reshape(1, NATOMS)
    cen = jnp.asarray(_CENTERS)
    cee = jnp.asarray(_CENTERS_EE)
    cena = jnp.asarray(_CENTERS_EN)
    eeWoT = ee_Wout.reshape(1, FEAT)
    enWoT = en_Wout.reshape(1, FEAT)
    enGA = en_edge_emb[0::2]
    enGE = en_edge_emb[1::2]

    grid = nb // BB
    full = lambda shape: pl.BlockSpec(shape, lambda i, s=len(shape): (0,) * s)
    out = pl.pallas_call(
        _body,
        grid=(grid,),
        in_specs=[
            pl.BlockSpec((BB, NELEC), lambda i: (i, 0)),
            pl.BlockSpec((BB, NELEC), lambda i: (i, 0)),
            pl.BlockSpec((BB, NELEC), lambda i: (i, 0)),
            full((1, NATOMS)), full((1, NATOMS)), full((1, NATOMS)),
            full((1, NRBF)),
            full((1, NRBF * NELEC)), full((1, NRBF * NATOMS)),
            full((2, FEAT)), full((3, FEAT)),
            full((NLAYERS, NRBF, FEAT)), full((NLAYERS, FEAT, FEAT)),
            full((1, FEAT)),
            full((2 + NATOMS, FEAT)),
            full((NATOMS, FEAT)), full((NATOMS, FEAT)),
            full((NLAYERS, NRBF, FEAT)), full((NLAYERS, FEAT, FEAT)),
            full((1, FEAT)),
        ],
        out_specs=pl.BlockSpec((1, 1, BB), lambda i: (i, 0, 0)),
        out_shape=jax.ShapeDtypeStruct((grid, 1, BB), jnp.float32),
        compiler_params=pltpu.CompilerParams(
            dimension_semantics=("parallel",)),
    )(X, Y, Z, AX, AY, AZ, cen, cee, cena,
      ee_node_emb, ee_edge_emb, ee_Wrbf, ee_Wself, eeWoT,
      en_node_emb, enGA, enGE, en_Wrbf, en_Wself, enWoT)
    return out.reshape(nb, 1)


# speedup vs baseline: 149.9263x; 1.0028x over previous
import numpy as np
import jax
import jax.numpy as jnp
from jax import lax
from jax.experimental import pallas as pl
from jax.experimental.pallas import tpu as pltpu

NELEC = 64
NUP = 32
NATOMS = 16
FEAT = 32
NRBF = 32
NLAYERS = 2
NBATCH = 256
GAMMA = 10.0
BB = 32

_CENTERS = np.linspace(0.0, 10.0, NRBF).astype(np.float32).reshape(1, NRBF)
_CENTERS_EE = np.repeat(_CENTERS.ravel(), NELEC).reshape(1, NRBF * NELEC)
_CENTERS_EN = np.repeat(_CENTERS.ravel(), NATOMS).reshape(1, NRBF * NATOMS)


def _body(x_ref, y_ref, z_ref, ax_ref, ay_ref, az_ref, cen_ref,
          cee_ref, cen_a_ref,
          een_ref, eeg_ref, eewr_ref, eews_ref, eewo_ref,
          enn_ref, enga_ref, enge_ref, enwr_ref, enws_ref, enwo_ref,
          out_ref):
    x = x_ref[...]
    y = y_ref[...]
    z = z_ref[...]
    cen = cen_ref[...]
    rbf0 = jnp.exp(-GAMMA * (1e-6 - cen) ** 2)

    jmask = lax.broadcasted_iota(jnp.int32, (NELEC, FEAT), 0) >= NUP

    dx = x[:, :, None] - x[:, None, :]
    dy = y[:, :, None] - y[:, None, :]
    dz = z[:, :, None] - z[:, None, :]
    dee = jnp.sqrt(dx * dx + dy * dy + dz * dz + 1e-12)
    D2 = jnp.concatenate([dee] * NRBF, axis=2)
    R2 = jnp.exp(-GAMMA * (D2 - cee_ref[...].reshape(1, 1, NRBF * NELEC)) ** 2
                 ).astype(jnp.bfloat16)

    eeg = eeg_ref[...]
    g_up = jnp.where(jmask, eeg[1:2, :], eeg[0:1, :])
    g_dn = jnp.where(jmask, eeg[2:3, :], eeg[1:2, :])

    een = een_ref[...]
    h0 = jnp.where(jmask, een[1:2, :], een[0:1, :])

    h = jnp.broadcast_to(h0[None], (BB, NELEC, FEAT))
    for l in range(NLAYERS):
        Wcat = jnp.concatenate([eewr_ref[l], eewr_ref[l]], axis=1)
        rbf0W = jnp.sum(rbf0.reshape(NRBF, 1) * Wcat, axis=0,
                        keepdims=True)
        if l == 0:
            HG0 = jnp.concatenate([g_up * h0, g_dn * h0], axis=1)
            B2 = (HG0[None, :, :] * Wcat[:, None, :]
                  ).reshape(NRBF * NELEC, 2 * FEAT).astype(jnp.bfloat16)
            AG = lax.dot_general(R2, B2, (((2,), (0,)), ((), ())),
                                 preferred_element_type=jnp.float32)
            AG = AG - (rbf0W * HG0)[None]
        else:
            HG = jnp.concatenate([g_up[None] * h, g_dn[None] * h], axis=2)
            B2 = (HG.astype(jnp.bfloat16)[:, None, :, :]
                  * Wcat.astype(jnp.bfloat16).reshape(1, NRBF, 1, 2 * FEAT)
                  ).reshape(BB, NRBF * NELEC, 2 * FEAT)
            AG = lax.dot_general(R2, B2, (((2,), (1,)), ((0,), (0,))),
                                 preferred_element_type=jnp.float32)
            AG = AG - rbf0W[None] * HG
        agg = jnp.concatenate([AG[:, :NUP, :FEAT], AG[:, NUP:, FEAT:]],
                              axis=1)
        upd = jnp.dot(agg.reshape(BB * NELEC, FEAT), eews_ref[l],
                      preferred_element_type=jnp.float32)
        h = h + jnp.tanh(upd).reshape(BB, NELEC, FEAT)

    ee_k = jnp.sum(jnp.sum(h, axis=1) * eewo_ref[...], axis=1)

    axr = ax_ref[...].reshape(1, 1, NATOMS)
    ayr = ay_ref[...].reshape(1, 1, NATOMS)
    azr = az_ref[...].reshape(1, 1, NATOMS)
    dxa = x[:, :, None] - axr
    dya = y[:, :, None] - ayr
    dza = z[:, :, None] - azr
    den_ei = jnp.sqrt(dxa * dxa + dya * dya + dza * dza + 1e-12)
    D2e = jnp.concatenate([den_ei] * NRBF, axis=2)
    Ren_e = jnp.exp(
        -GAMMA * (D2e - cen_a_ref[...].reshape(1, 1, NRBF * NATOMS)) ** 2
    ).astype(jnp.bfloat16)

    dxb = ax_ref[...].reshape(1, NATOMS, 1) - x[:, None, :]
    dyb = ay_ref[...].reshape(1, NATOMS, 1) - y[:, None, :]
    dzb = az_ref[...].reshape(1, NATOMS, 1) - z[:, None, :]
    den_ai = jnp.sqrt(dxb * dxb + dyb * dyb + dzb * dzb + 1e-12)
    ckk = cen.reshape(1, NRBF, 1, 1)
    Ren_a = jnp.exp(-GAMMA * (den_ai[:, None, :, :] - ckk) ** 2
                    ).reshape(BB, NRBF * NATOMS, NELEC
                              ).astype(jnp.bfloat16)

    enn = enn_ref[...]
    he0 = jnp.where(jmask, enn[1:2, :], enn[0:1, :])
    ha0 = enn[2:2 + NATOMS]
    he = jnp.broadcast_to(he0[None], (BB, NELEC, FEAT))
    ha = jnp.broadcast_to(ha0[None], (BB, NATOMS, FEAT))
    ga = enga_ref[...]
    ge = enge_ref[...]
    for l in range(NLAYERS):
        W = enwr_ref[l]
        if l == 0:
            Ben = (ge * ha0)[None, :, :] * W[:, None, :]
            agg_e = lax.dot_general(
                Ren_e, Ben.reshape(NRBF * NATOMS, FEAT).astype(jnp.bfloat16),
                (((2,), (0,)), ((), ())),
                preferred_element_type=jnp.float32)
            U = lax.dot_general(Ren_a, he0.astype(jnp.bfloat16),
                                (((2,), (0,)), ((), ())),
                                preferred_element_type=jnp.float32)
        else:
            gha = ge[None] * ha
            Ben = (gha.astype(jnp.bfloat16)[:, None, :, :]
                   * W.astype(jnp.bfloat16).reshape(1, NRBF, 1, FEAT)
                   ).reshape(BB, NRBF * NATOMS, FEAT)
            agg_e = lax.dot_general(Ren_e, Ben, (((2,), (1,)), ((0,), (0,))),
                                    preferred_element_type=jnp.float32)
            U = lax.dot_general(Ren_a, he.astype(jnp.bfloat16),
                                (((2,), (1,)), ((0,), (0,))),
                                preferred_element_type=jnp.float32)
        Ur = U.reshape(BB, NRBF, NATOMS, FEAT) * W.reshape(1, NRBF, 1, FEAT)
        agg_a = ga[None] * jnp.sum(Ur, axis=1)
        upd_e = jnp.dot(agg_e.reshape(BB * NELEC, FEAT), enws_ref[l],
                        preferred_element_type=jnp.float32)
        upd_a = jnp.dot(agg_a.reshape(BB * NATOMS, FEAT), enws_ref[l],
                        preferred_element_type=jnp.float32)
        he = he + jnp.tanh(upd_e).reshape(BB, NELEC, FEAT)
        ha = ha + jnp.tanh(upd_a).reshape(BB, NATOMS, FEAT)

    nsum = jnp.sum(he, axis=1) + jnp.sum(ha, axis=1)
    en_k = jnp.sum(nsum * enwo_ref[...], axis=1)

    out_ref[...] = jnp.exp(ee_k + en_k).reshape(1, 1, BB)


def kernel(pos, atom_coords, ee_node_emb, ee_edge_emb, ee_Wrbf, ee_Wself,
           ee_Wout, en_node_emb, en_edge_emb, en_Wrbf, en_Wself, en_Wout):
    nb = pos.shape[0]
    xyz = pos.reshape(nb, NELEC, 3)
    X = xyz[:, :, 0]
    Y = xyz[:, :, 1]
    Z = xyz[:, :, 2]
    AX = atom_coords[:, 0].reshape(1, NATOMS)
    AY = atom_coords[:, 1].reshape(1, NATOMS)
    AZ = atom_coords[:, 2].reshape(1, NATOMS)
    cen = jnp.asarray(_CENTERS)
    cee = jnp.asarray(_CENTERS_EE)
    cena = jnp.asarray(_CENTERS_EN)
    eeWoT = ee_Wout.reshape(1, FEAT)
    enWoT = en_Wout.reshape(1, FEAT)
    enGA = en_edge_emb[0::2]
    enGE = en_edge_emb[1::2]

    grid = nb // BB
    full = lambda shape: pl.BlockSpec(shape, lambda i, s=len(shape): (0,) * s)
    out = pl.pallas_call(
        _body,
        grid=(grid,),
        in_specs=[
            pl.BlockSpec((BB, NELEC), lambda i: (i, 0)),
            pl.BlockSpec((BB, NELEC), lambda i: (i, 0)),
            pl.BlockSpec((BB, NELEC), lambda i: (i, 0)),
            full((1, NATOMS)), full((1, NATOMS)), full((1, NATOMS)),
            full((1, NRBF)),
            full((1, NRBF * NELEC)), full((1, NRBF * NATOMS)),
            full((2, FEAT)), full((3, FEAT)),
            full((NLAYERS, NRBF, FEAT)), full((NLAYERS, FEAT, FEAT)),
            full((1, FEAT)),
            full((2 + NATOMS, FEAT)),
            full((NATOMS, FEAT)), full((NATOMS, FEAT)),
            full((NLAYERS, NRBF, FEAT)), full((NLAYERS, FEAT, FEAT)),
            full((1, FEAT)),
        ],
        out_specs=pl.BlockSpec((1, 1, BB), lambda i: (i, 0, 0)),
        out_shape=jax.ShapeDtypeStruct((grid, 1, BB), jnp.float32),
        compiler_params=pltpu.CompilerParams(
            dimension_semantics=("parallel",)),
    )(X, Y, Z, AX, AY, AZ, cen, cee, cena,
      ee_node_emb, ee_edge_emb, ee_Wrbf, ee_Wself, eeWoT,
      en_node_emb, enGA, enGE, en_Wrbf, en_Wself, enWoT)
    return out.reshape(nb, 1)
